# baseline (speedup 1.0000x reference)
import functools

import jax
import jax.numpy as jnp
from jax import lax
from jax.experimental import pallas as pl
from jax.experimental.pallas import tpu as pltpu
from jax.experimental.pallas import tpu_sc as plsc

N = 262144
B = 4096
H = 128

BLK = 16384
NBLK = N // BLK

try:
    _info = plsc.get_sparse_core_info()
    NC = _info.num_cores
    NS = _info.num_subcores
except Exception:
    NC = 2
    NS = 16
NW = NC * NS
CHUNK = N // NW


def _stage1_body(aux_ref, ph_ref, eh_ref, lat_ref, r_ref, errl_ref):
    pid = pl.program_id(0)
    d = eh_ref[...] - ph_ref[...]
    rh = jax.lax.dot_general(
        jnp.ones((1, H), jnp.float32), d * d,
        dimension_numbers=(((1,), (1,)), ((), ())),
        preferred_element_type=jnp.float32,
    )
    sig = aux_ref[6:7, :]
    sig2 = sig * sig
    rx = jnp.zeros((1, BLK), jnp.float32)
    for c in range(3):
        e = aux_ref[c:c + 1, :] / sig2 - aux_ref[3 + c:4 + c, :]
        rx = rx + e * e
    r_ref[...] = (rh + 0.5 * sig2 * rx).reshape(BLK)

    @pl.when(pid == 0)
    def _():
        tot = jnp.zeros((B,), jnp.float32)
        for i in range(3):
            for j in range(3):
                acc = jnp.zeros((B,), jnp.float32)
                for k in range(3):
                    acc = acc + lat_ref[3 * i + k, :] * lat_ref[9 + 3 * k + j, :]
                dlt = lat_ref[18 + 3 * i + j, :] - acc
                tot = tot + dlt * dlt
        errl_ref[0, 0] = jnp.sum(tot)


_stage1 = pl.pallas_call(
    _stage1_body,
    grid=(NBLK,),
    in_specs=[
        pl.BlockSpec((8, BLK), lambda i: (0, i)),
        pl.BlockSpec((BLK, H), lambda i: (i, 0)),
        pl.BlockSpec((BLK, H), lambda i: (i, 0)),
        pl.BlockSpec((32, B), lambda i: (0, 0)),
    ],
    out_specs=[
        pl.BlockSpec((BLK,), lambda i: (i,)),
        pl.BlockSpec((1, 1), lambda i: (0, 0), memory_space=pltpu.SMEM),
    ],
    out_shape=[
        jax.ShapeDtypeStruct((N,), jnp.float32),
        jax.ShapeDtypeStruct((1, 1), jnp.float32),
    ],
    compiler_params=pltpu.CompilerParams(
        dimension_semantics=("arbitrary",),
    ),
)


def _sc_body(ids_hbm, r_hbm, sums_hbm, cnts_hbm, idx_v, val_v, sums_v, cnts_v):
    cid = lax.axis_index("c")
    sid = lax.axis_index("s")
    wid = cid * NS + sid
    base = wid * CHUNK
    pltpu.sync_copy(ids_hbm.at[pl.ds(base, CHUNK)], idx_v)
    pltpu.sync_copy(r_hbm.at[pl.ds(base, CHUNK)], val_v)

    zeros16 = jnp.zeros((16,), jnp.float32)

    def zero_body(i, carry):
        sums_v[pl.ds(i * 16, 16)] = zeros16
        cnts_v[pl.ds(i * 16, 16)] = zeros16
        return carry

    lax.fori_loop(0, B // 16, zero_body, 0)

    ones16 = jnp.full((16,), 1.0, jnp.float32)

    def scat_body(i, carry):
        ix = idx_v[pl.ds(i * 16, 16)]
        v = val_v[pl.ds(i * 16, 16)]
        plsc.addupdate_scatter(sums_v, [ix], v)
        plsc.addupdate_scatter(cnts_v, [ix], ones16)
        return carry

    lax.fori_loop(0, CHUNK // 16, scat_body, 0)

    pltpu.sync_copy(sums_v, sums_hbm.at[wid])
    pltpu.sync_copy(cnts_v, cnts_hbm.at[wid])


@functools.lru_cache(maxsize=1)
def _sc_scatter():
    return functools.partial(
        pl.kernel,
        mesh=plsc.VectorSubcoreMesh(core_axis_name="c", subcore_axis_name="s",
                                    num_cores=NC, num_subcores=NS),
        out_type=[
            jax.ShapeDtypeStruct((NW, B), jnp.float32),
            jax.ShapeDtypeStruct((NW, B), jnp.float32),
        ],
        scratch_types=[
            pltpu.VMEM((CHUNK,), jnp.int32),
            pltpu.VMEM((CHUNK,), jnp.float32),
            pltpu.VMEM((B,), jnp.float32),
            pltpu.VMEM((B,), jnp.float32),
        ],
        compiler_params=pltpu.CompilerParams(needs_layout_passes=False),
    )(_sc_body)


def _stage3_body(sums_ref, cnts_ref, errl_ref, out_ref):
    s = jnp.sum(sums_ref[...], axis=0)
    c = jnp.sum(cnts_ref[...], axis=0)
    seg = s / jnp.maximum(c, 1.0)
    out_ref[0, 0] = jnp.sum(seg) * (1.0 / B) + errl_ref[0, 0]


_stage3 = pl.pallas_call(
    _stage3_body,
    in_specs=[
        pl.BlockSpec((NW, B), lambda: (0, 0)),
        pl.BlockSpec((NW, B), lambda: (0, 0)),
        pl.BlockSpec((1, 1), lambda: (0, 0), memory_space=pltpu.SMEM),
    ],
    out_specs=pl.BlockSpec((1, 1), lambda: (0, 0), memory_space=pltpu.SMEM),
    out_shape=jax.ShapeDtypeStruct((1, 1), jnp.float32),
)


def kernel(pred_eps_x, target_eps_x, used_sigmas_x, pred_eps_h, eps_h,
           pred_eps_l, eps_l, inv_rot_mat, batch_ids):
    aux = jnp.concatenate(
        [target_eps_x.T, pred_eps_x.T, used_sigmas_x.T,
         jnp.zeros((1, N), jnp.float32)], axis=0)
    lat = jnp.concatenate(
        [inv_rot_mat.reshape(B, 9).T, pred_eps_l.reshape(B, 9).T,
         eps_l.reshape(B, 9).T, jnp.zeros((5, B), jnp.float32)], axis=0)
    r, errl = _stage1(aux, pred_eps_h, eps_h, lat)
    sums, cnts = _sc_scatter()(batch_ids, r)
    out = _stage3(sums, cnts, errl)
    return out[0, 0]


# speedup vs baseline: 1.0176x; 1.0176x over previous
import functools

import jax
import jax.numpy as jnp
from jax import lax
from jax.experimental import pallas as pl
from jax.experimental.pallas import tpu as pltpu
from jax.experimental.pallas import tpu_sc as plsc

N = 262144
B = 4096
H = 128

BLK = 8192
NBLK = N // BLK

try:
    _info = plsc.get_sparse_core_info()
    NC = _info.num_cores
    NS = _info.num_subcores
except Exception:
    NC = 2
    NS = 16
NW = NC * NS
CHUNK = N // NW


def _stage1_body(aux_ref, ph_ref, eh_ref, lat_ref, r_ref, errl_ref):
    pid = pl.program_id(0)
    d = eh_ref[...] - ph_ref[...]
    rh = jax.lax.dot_general(
        jnp.ones((1, H), jnp.float32), d * d,
        dimension_numbers=(((1,), (1,)), ((), ())),
        preferred_element_type=jnp.float32,
    )
    sig = aux_ref[6:7, :]
    sig2 = sig * sig
    rx = jnp.zeros((1, BLK), jnp.float32)
    for c in range(3):
        e = aux_ref[c:c + 1, :] / sig2 - aux_ref[3 + c:4 + c, :]
        rx = rx + e * e
    r_ref[...] = (rh + 0.5 * sig2 * rx).reshape(BLK)

    @pl.when(pid == 0)
    def _():
        tot = jnp.zeros((B,), jnp.float32)
        for i in range(3):
            for j in range(3):
                acc = jnp.zeros((B,), jnp.float32)
                for k in range(3):
                    acc = acc + lat_ref[3 * i + k, :] * lat_ref[9 + 3 * k + j, :]
                dlt = lat_ref[18 + 3 * i + j, :] - acc
                tot = tot + dlt * dlt
        errl_ref[0, 0] = jnp.sum(tot)


_stage1 = pl.pallas_call(
    _stage1_body,
    grid=(NBLK,),
    in_specs=[
        pl.BlockSpec((8, BLK), lambda i: (0, i)),
        pl.BlockSpec((BLK, H), lambda i: (i, 0)),
        pl.BlockSpec((BLK, H), lambda i: (i, 0)),
        pl.BlockSpec((32, B), lambda i: (0, 0)),
    ],
    out_specs=[
        pl.BlockSpec((BLK,), lambda i: (i,)),
        pl.BlockSpec((1, 1), lambda i: (0, 0), memory_space=pltpu.SMEM),
    ],
    out_shape=[
        jax.ShapeDtypeStruct((N,), jnp.float32),
        jax.ShapeDtypeStruct((1, 1), jnp.float32),
    ],
    compiler_params=pltpu.CompilerParams(
        dimension_semantics=("arbitrary",),
    ),
)


def _sc_body(ids_hbm, r_hbm, sums_hbm, cnts_hbm, idx_v, val_v, sums_v, cnts_v):
    cid = lax.axis_index("c")
    sid = lax.axis_index("s")
    wid = cid * NS + sid
    base = wid * CHUNK
    pltpu.sync_copy(ids_hbm.at[pl.ds(base, CHUNK)], idx_v)
    pltpu.sync_copy(r_hbm.at[pl.ds(base, CHUNK)], val_v)

    zeros16 = jnp.zeros((16,), jnp.float32)

    def zero_body(i, carry):
        for u in range(4):
            off = (i * 4 + u) * 16
            sums_v[pl.ds(off, 16)] = zeros16
            cnts_v[pl.ds(off, 16)] = zeros16
        return carry

    lax.fori_loop(0, B // 64, zero_body, 0)

    ones16 = jnp.full((16,), 1.0, jnp.float32)

    def scat_body(i, carry):
        for u in range(4):
            off = (i * 4 + u) * 16
            ix = idx_v[pl.ds(off, 16)]
            v = val_v[pl.ds(off, 16)]
            plsc.addupdate_scatter(sums_v, [ix], v)
            plsc.addupdate_scatter(cnts_v, [ix], ones16)
        return carry

    lax.fori_loop(0, CHUNK // 64, scat_body, 0)

    pltpu.sync_copy(sums_v, sums_hbm.at[wid])
    pltpu.sync_copy(cnts_v, cnts_hbm.at[wid])


@functools.lru_cache(maxsize=1)
def _sc_scatter():
    return functools.partial(
        pl.kernel,
        mesh=plsc.VectorSubcoreMesh(core_axis_name="c", subcore_axis_name="s",
                                    num_cores=NC, num_subcores=NS),
        out_type=[
            jax.ShapeDtypeStruct((NW, B), jnp.float32),
            jax.ShapeDtypeStruct((NW, B), jnp.float32),
        ],
        scratch_types=[
            pltpu.VMEM((CHUNK,), jnp.int32),
            pltpu.VMEM((CHUNK,), jnp.float32),
            pltpu.VMEM((B,), jnp.float32),
            pltpu.VMEM((B,), jnp.float32),
        ],
        compiler_params=pltpu.CompilerParams(needs_layout_passes=False),
    )(_sc_body)


def _stage3_body(sums_ref, cnts_ref, errl_ref, out_ref):
    s = jnp.sum(sums_ref[...], axis=0)
    c = jnp.sum(cnts_ref[...], axis=0)
    seg = s / jnp.maximum(c, 1.0)
    out_ref[0, 0] = jnp.sum(seg) * (1.0 / B) + errl_ref[0, 0]


_stage3 = pl.pallas_call(
    _stage3_body,
    in_specs=[
        pl.BlockSpec((NW, B), lambda: (0, 0)),
        pl.BlockSpec((NW, B), lambda: (0, 0)),
        pl.BlockSpec((1, 1), lambda: (0, 0), memory_space=pltpu.SMEM),
    ],
    out_specs=pl.BlockSpec((1, 1), lambda: (0, 0), memory_space=pltpu.SMEM),
    out_shape=jax.ShapeDtypeStruct((1, 1), jnp.float32),
)


def kernel(pred_eps_x, target_eps_x, used_sigmas_x, pred_eps_h, eps_h,
           pred_eps_l, eps_l, inv_rot_mat, batch_ids):
    aux = jnp.concatenate(
        [target_eps_x.T, pred_eps_x.T, used_sigmas_x.T,
         jnp.zeros((1, N), jnp.float32)], axis=0)
    lat = jnp.concatenate(
        [inv_rot_mat.reshape(B, 9).T, pred_eps_l.reshape(B, 9).T,
         eps_l.reshape(B, 9).T, jnp.zeros((5, B), jnp.float32)], axis=0)
    r, errl = _stage1(aux, pred_eps_h, eps_h, lat)
    sums, cnts = _sc_scatter()(batch_ids, r)
    out = _stage3(sums, cnts, errl)
    return out[0, 0]


# speedup vs baseline: 1.0384x; 1.0205x over previous
import functools

import jax
import jax.numpy as jnp
from jax import lax
from jax.experimental import pallas as pl
from jax.experimental.pallas import tpu as pltpu
from jax.experimental.pallas import tpu_sc as plsc

N = 262144
B = 4096
H = 128

BLK = 8192
NBLK = N // BLK
NH = N // 2
HBLKS = NH // BLK

try:
    _info = plsc.get_sparse_core_info()
    NC = _info.num_cores
    NS = _info.num_subcores
except Exception:
    NC = 2
    NS = 16
NW = NC * NS
CHUNKH = NH // NW


def _make_stage1(off_blocks, with_lat):
    def body(aux_ref, ph_ref, eh_ref, lat_ref, r_ref, errl_ref):
        pid = pl.program_id(0)
        d = eh_ref[...] - ph_ref[...]
        rh = jax.lax.dot_general(
            jnp.ones((1, H), jnp.float32), d * d,
            dimension_numbers=(((1,), (1,)), ((), ())),
            preferred_element_type=jnp.float32,
        )
        sig = aux_ref[6:7, :]
        sig2 = sig * sig
        rx = jnp.zeros((1, BLK), jnp.float32)
        for c in range(3):
            e = aux_ref[c:c + 1, :] / sig2 - aux_ref[3 + c:4 + c, :]
            rx = rx + e * e
        r_ref[...] = (rh + 0.5 * sig2 * rx).reshape(BLK)

        @pl.when(pid == 0)
        def _():
            if with_lat:
                tot = jnp.zeros((B,), jnp.float32)
                for i in range(3):
                    for j in range(3):
                        acc = jnp.zeros((B,), jnp.float32)
                        for k in range(3):
                            acc = acc + (lat_ref[3 * i + k, :]
                                         * lat_ref[9 + 3 * k + j, :])
                        dlt = lat_ref[18 + 3 * i + j, :] - acc
                        tot = tot + dlt * dlt
                errl_ref[0, 0] = jnp.sum(tot)
            else:
                errl_ref[0, 0] = 0.0

    return pl.pallas_call(
        body,
        grid=(HBLKS,),
        in_specs=[
            pl.BlockSpec((8, BLK), lambda i: (0, i + off_blocks)),
            pl.BlockSpec((BLK, H), lambda i: (i + off_blocks, 0)),
            pl.BlockSpec((BLK, H), lambda i: (i + off_blocks, 0)),
            pl.BlockSpec((32, B), lambda i: (0, 0)),
        ],
        out_specs=[
            pl.BlockSpec((BLK,), lambda i: (i,)),
            pl.BlockSpec((1, 1), lambda i: (0, 0), memory_space=pltpu.SMEM),
        ],
        out_shape=[
            jax.ShapeDtypeStruct((NH,), jnp.float32),
            jax.ShapeDtypeStruct((1, 1), jnp.float32),
        ],
        compiler_params=pltpu.CompilerParams(
            dimension_semantics=("arbitrary",),
        ),
    )


_stage1_a = _make_stage1(0, True)
_stage1_b = _make_stage1(HBLKS, False)


def _make_sc_body(off):
    def _sc_body(ids_hbm, r_hbm, sums_hbm, cnts_hbm,
                 idx_v, val_v, sums_v, cnts_v):
        cid = lax.axis_index("c")
        sid = lax.axis_index("s")
        wid = cid * NS + sid
        base = wid * CHUNKH
        pltpu.sync_copy(ids_hbm.at[pl.ds(off + base, CHUNKH)], idx_v)
        pltpu.sync_copy(r_hbm.at[pl.ds(base, CHUNKH)], val_v)

        zeros16 = jnp.zeros((16,), jnp.float32)

        def zero_body(i, carry):
            for u in range(4):
                o = (i * 4 + u) * 16
                sums_v[pl.ds(o, 16)] = zeros16
                cnts_v[pl.ds(o, 16)] = zeros16
            return carry

        lax.fori_loop(0, B // 64, zero_body, 0)

        ones16 = jnp.full((16,), 1.0, jnp.float32)

        def scat_body(i, carry):
            for u in range(4):
                o = (i * 4 + u) * 16
                ix = idx_v[pl.ds(o, 16)]
                v = val_v[pl.ds(o, 16)]
                plsc.addupdate_scatter(sums_v, [ix], v)
                plsc.addupdate_scatter(cnts_v, [ix], ones16)
            return carry

        lax.fori_loop(0, CHUNKH // 64, scat_body, 0)

        pltpu.sync_copy(sums_v, sums_hbm.at[wid])
        pltpu.sync_copy(cnts_v, cnts_hbm.at[wid])

    return _sc_body


@functools.lru_cache(maxsize=2)
def _sc_scatter(off):
    return functools.partial(
        pl.kernel,
        mesh=plsc.VectorSubcoreMesh(core_axis_name="c", subcore_axis_name="s",
                                    num_cores=NC, num_subcores=NS),
        out_type=[
            jax.ShapeDtypeStruct((NW, B), jnp.float32),
            jax.ShapeDtypeStruct((NW, B), jnp.float32),
        ],
        scratch_types=[
            pltpu.VMEM((CHUNKH,), jnp.int32),
            pltpu.VMEM((CHUNKH,), jnp.float32),
            pltpu.VMEM((B,), jnp.float32),
            pltpu.VMEM((B,), jnp.float32),
        ],
        compiler_params=pltpu.CompilerParams(needs_layout_passes=False),
    )(_make_sc_body(off))


def _stage3_body(sa_ref, ca_ref, sb_ref, cb_ref, errl_ref, out_ref):
    s = jnp.sum(sa_ref[...], axis=0) + jnp.sum(sb_ref[...], axis=0)
    c = jnp.sum(ca_ref[...], axis=0) + jnp.sum(cb_ref[...], axis=0)
    seg = s / jnp.maximum(c, 1.0)
    out_ref[0, 0] = jnp.sum(seg) * (1.0 / B) + errl_ref[0, 0]


_stage3 = pl.pallas_call(
    _stage3_body,
    in_specs=[
        pl.BlockSpec((NW, B), lambda: (0, 0)),
        pl.BlockSpec((NW, B), lambda: (0, 0)),
        pl.BlockSpec((NW, B), lambda: (0, 0)),
        pl.BlockSpec((NW, B), lambda: (0, 0)),
        pl.BlockSpec((1, 1), lambda: (0, 0), memory_space=pltpu.SMEM),
    ],
    out_specs=pl.BlockSpec((1, 1), lambda: (0, 0), memory_space=pltpu.SMEM),
    out_shape=jax.ShapeDtypeStruct((1, 1), jnp.float32),
)


def kernel(pred_eps_x, target_eps_x, used_sigmas_x, pred_eps_h, eps_h,
           pred_eps_l, eps_l, inv_rot_mat, batch_ids):
    aux = jnp.concatenate(
        [target_eps_x.T, pred_eps_x.T, used_sigmas_x.T,
         jnp.zeros((1, N), jnp.float32)], axis=0)
    lat = jnp.concatenate(
        [inv_rot_mat.reshape(B, 9).T, pred_eps_l.reshape(B, 9).T,
         eps_l.reshape(B, 9).T, jnp.zeros((5, B), jnp.float32)], axis=0)
    r_a, errl = _stage1_a(aux, pred_eps_h, eps_h, lat)
    sums_a, cnts_a = _sc_scatter(0)(batch_ids, r_a)
    r_b, _ = _stage1_b(aux, pred_eps_h, eps_h, lat)
    sums_b, cnts_b = _sc_scatter(NH)(batch_ids, r_b)
    out = _stage3(sums_a, cnts_a, sums_b, cnts_b, errl)
    return out[0, 0]


# speedup vs baseline: 1.1017x; 1.0609x over previous
import functools

import jax
import jax.numpy as jnp
from jax import lax
from jax.experimental import pallas as pl
from jax.experimental.pallas import tpu as pltpu
from jax.experimental.pallas import tpu_sc as plsc

N = 262144
B = 4096
H = 128

BLK = 8192
NBLK = N // BLK
ABLKS = 24
BBLKS = NBLK - ABLKS
NA = ABLKS * BLK
NB = BBLKS * BLK

try:
    _info = plsc.get_sparse_core_info()
    NC = _info.num_cores
    NS = _info.num_subcores
except Exception:
    NC = 2
    NS = 16
NW = NC * NS


def _make_stage1(off_blocks, nblocks, with_lat):
    def body(aux_ref, ph_ref, eh_ref, lat_ref, r_ref, errl_ref):
        pid = pl.program_id(0)
        d = eh_ref[...] - ph_ref[...]
        rh = jax.lax.dot_general(
            jnp.ones((1, H), jnp.float32), d * d,
            dimension_numbers=(((1,), (1,)), ((), ())),
            preferred_element_type=jnp.float32,
        )
        sig = aux_ref[6:7, :]
        sig2 = sig * sig
        rx = jnp.zeros((1, BLK), jnp.float32)
        for c in range(3):
            e = aux_ref[c:c + 1, :] / sig2 - aux_ref[3 + c:4 + c, :]
            rx = rx + e * e
        r_ref[...] = (rh + 0.5 * sig2 * rx).reshape(BLK)

        @pl.when(pid == 0)
        def _():
            if with_lat:
                tot = jnp.zeros((B,), jnp.float32)
                for i in range(3):
                    for j in range(3):
                        acc = jnp.zeros((B,), jnp.float32)
                        for k in range(3):
                            acc = acc + (lat_ref[3 * i + k, :]
                                         * lat_ref[9 + 3 * k + j, :])
                        dlt = lat_ref[18 + 3 * i + j, :] - acc
                        tot = tot + dlt * dlt
                errl_ref[0, 0] = jnp.sum(tot)
            else:
                errl_ref[0, 0] = 0.0

    return pl.pallas_call(
        body,
        grid=(nblocks,),
        in_specs=[
            pl.BlockSpec((8, BLK), lambda i: (0, i + off_blocks)),
            pl.BlockSpec((BLK, H), lambda i: (i + off_blocks, 0)),
            pl.BlockSpec((BLK, H), lambda i: (i + off_blocks, 0)),
            pl.BlockSpec((32, B), lambda i: (0, 0)),
        ],
        out_specs=[
            pl.BlockSpec((BLK,), lambda i: (i,)),
            pl.BlockSpec((1, 1), lambda i: (0, 0), memory_space=pltpu.SMEM),
        ],
        out_shape=[
            jax.ShapeDtypeStruct((nblocks * BLK,), jnp.float32),
            jax.ShapeDtypeStruct((1, 1), jnp.float32),
        ],
        compiler_params=pltpu.CompilerParams(
            dimension_semantics=("arbitrary",),
        ),
    )


_stage1_a = _make_stage1(0, ABLKS, True)
_stage1_b = _make_stage1(ABLKS, BBLKS, False)


def _make_sc_body(off, chunk):
    def _sc_body(ids_hbm, r_hbm, sums_hbm, cnts_hbm,
                 idx_v, val_v, sums_v, cnts_v):
        cid = lax.axis_index("c")
        sid = lax.axis_index("s")
        wid = cid * NS + sid
        base = wid * chunk
        pltpu.sync_copy(ids_hbm.at[pl.ds(off + base, chunk)], idx_v)
        pltpu.sync_copy(r_hbm.at[pl.ds(base, chunk)], val_v)

        zeros16 = jnp.zeros((16,), jnp.float32)

        def zero_body(i, carry):
            for u in range(4):
                o = (i * 4 + u) * 16
                sums_v[pl.ds(o, 16)] = zeros16
                cnts_v[pl.ds(o, 16)] = zeros16
            return carry

        lax.fori_loop(0, B // 64, zero_body, 0)

        ones16 = jnp.full((16,), 1.0, jnp.float32)

        def scat_body(i, carry):
            for u in range(4):
                o = (i * 4 + u) * 16
                ix = idx_v[pl.ds(o, 16)]
                v = val_v[pl.ds(o, 16)]
                plsc.addupdate_scatter(sums_v, [ix], v)
                plsc.addupdate_scatter(cnts_v, [ix], ones16)
            return carry

        lax.fori_loop(0, chunk // 64, scat_body, 0)

        pltpu.sync_copy(sums_v, sums_hbm.at[wid])
        pltpu.sync_copy(cnts_v, cnts_hbm.at[wid])

    return _sc_body


@functools.lru_cache(maxsize=2)
def _sc_scatter(off, chunk):
    return functools.partial(
        pl.kernel,
        mesh=plsc.VectorSubcoreMesh(core_axis_name="c", subcore_axis_name="s",
                                    num_cores=NC, num_subcores=NS),
        out_type=[
            jax.ShapeDtypeStruct((NW, B), jnp.float32),
            jax.ShapeDtypeStruct((NW, B), jnp.float32),
        ],
        scratch_types=[
            pltpu.VMEM((chunk,), jnp.int32),
            pltpu.VMEM((chunk,), jnp.float32),
            pltpu.VMEM((B,), jnp.float32),
            pltpu.VMEM((B,), jnp.float32),
        ],
        compiler_params=pltpu.CompilerParams(needs_layout_passes=False),
    )(_make_sc_body(off, chunk))


def _stage3_body(sa_ref, ca_ref, sb_ref, cb_ref, errl_ref, out_ref):
    s = jnp.sum(sa_ref[...], axis=0) + jnp.sum(sb_ref[...], axis=0)
    c = jnp.sum(ca_ref[...], axis=0) + jnp.sum(cb_ref[...], axis=0)
    seg = s / jnp.maximum(c, 1.0)
    out_ref[0, 0] = jnp.sum(seg) * (1.0 / B) + errl_ref[0, 0]


_stage3 = pl.pallas_call(
    _stage3_body,
    in_specs=[
        pl.BlockSpec((NW, B), lambda: (0, 0)),
        pl.BlockSpec((NW, B), lambda: (0, 0)),
        pl.BlockSpec((NW, B), lambda: (0, 0)),
        pl.BlockSpec((NW, B), lambda: (0, 0)),
        pl.BlockSpec((1, 1), lambda: (0, 0), memory_space=pltpu.SMEM),
    ],
    out_specs=pl.BlockSpec((1, 1), lambda: (0, 0), memory_space=pltpu.SMEM),
    out_shape=jax.ShapeDtypeStruct((1, 1), jnp.float32),
)


def kernel(pred_eps_x, target_eps_x, used_sigmas_x, pred_eps_h, eps_h,
           pred_eps_l, eps_l, inv_rot_mat, batch_ids):
    aux = jnp.concatenate(
        [target_eps_x.T, pred_eps_x.T, used_sigmas_x.T,
         jnp.zeros((1, N), jnp.float32)], axis=0)
    lat = jnp.concatenate(
        [inv_rot_mat.reshape(B, 9).T, pred_eps_l.reshape(B, 9).T,
         eps_l.reshape(B, 9).T, jnp.zeros((5, B), jnp.float32)], axis=0)
    r_a, errl = _stage1_a(aux, pred_eps_h, eps_h, lat)
    sums_a, cnts_a = _sc_scatter(0, NA // NW)(batch_ids, r_a)
    r_b, _ = _stage1_b(aux, pred_eps_h, eps_h, lat)
    sums_b, cnts_b = _sc_scatter(NA, NB // NW)(batch_ids, r_b)
    out = _stage3(sums_a, cnts_a, sums_b, cnts_b, errl)
    return out[0, 0]
